# baseline (speedup 1.0000x reference)
import jax
import jax.numpy as jnp
from jax import lax
from jax.experimental import pallas as pl
from jax.experimental.pallas import tpu as pltpu
from jax.experimental.pallas import tpu_sc as plsc

L = 16
NC, NS = 2, 16
NW = NC * NS
CHUNK = 1024


def _tap_table():
    taps = []
    for i in range(4):
        for j in range(4):
            lst = []
            for u in range(3):
                for v in range(3):
                    ii, jj = i + u - 1, j + v - 1
                    if 0 <= ii < 4 and 0 <= jj < 4:
                        lst.append((ii * 4 + jj, u * 3 + v))
            taps.append(lst)
    return taps


_TAPS = _tap_table()


def _sc_body(x_hbm, w_hbm, out_hbm, xa, xb, ya, yb, wv, wscr, sia, sib, soa, sob):
    c = lax.axis_index("c")
    s = lax.axis_index("s")
    wid = s * NC + c
    n = x_hbm.shape[0] // L
    tiles_per_worker = n // NW
    n_chunks = tiles_per_worker // CHUNK

    pltpu.sync_copy(w_hbm, wv)
    w16 = wv[...]

    def bcast_lane(vec, k):
        return lax.gather(
            vec,
            jnp.full((L, 1), k, jnp.int32),
            lax.GatherDimensionNumbers(
                offset_dims=(), collapsed_slice_dims=(0,), start_index_map=(0,)
            ),
            slice_sizes=(1,),
            mode=lax.GatherScatterMode.PROMISE_IN_BOUNDS,
        )

    wvecs = [bcast_lane(w16, k) for k in range(9)]
    for k in range(9):
        wscr[pl.ds(L * k, L)] = plsc.bitcast(
            plsc.pack(wvecs[k], wvecs[k], format=plsc.PackFormat.INTERLEAVED),
            jnp.float32,
        )
    zb = jnp.zeros((2 * L,), jnp.bfloat16)

    start = wid * tiles_per_worker

    def issue_in(ci, buf, s_in):
        base = start + ci * CHUNK
        for r in range(L):
            pltpu.async_copy(
                x_hbm.at[pl.ds(r * n + base, CHUNK)],
                buf.at[pl.ds(r * CHUNK, CHUNK)],
                s_in,
            )

    def drain_in(buf, s_in):
        pltpu.make_async_copy(x_hbm.at[pl.ds(0, L * CHUNK)], buf, s_in).wait()

    def issue_out(ci, buf, s_out):
        base = start + ci * CHUNK
        for r in range(L):
            pltpu.async_copy(
                buf.at[pl.ds(r * CHUNK, CHUNK)],
                out_hbm.at[pl.ds(r * n + base, CHUNK)],
                s_out,
            )

    def drain_out(buf, s_out):
        pltpu.make_async_copy(buf, out_hbm.at[pl.ds(0, L * CHUNK)], s_out).wait()

    def compute(buf_in, buf_out):
        @plsc.parallel_loop(0, CHUNK // (2 * L), 1, unroll=2)
        def group_body(g):
            off = g * (2 * L)
            xb = []
            for r in range(L):
                a = buf_in[pl.ds(r * CHUNK + off, L)]
                b = buf_in[pl.ds(r * CHUNK + off + L, L)]
                xb.append(plsc.pack(a, b, format=plsc.PackFormat.INTERLEAVED))
            wb = [plsc.bitcast(wscr[pl.ds(L * k, L)], jnp.bfloat16)
                  for k in range(9)]
            for r in range(L):
                acc = None
                for (rs, widx) in _TAPS[r]:
                    term = wb[widx] * xb[rs]
                    acc = term if acc is None else acc + term
                acc = jnp.where(xb[r] == zb, zb, acc)
                oa, ob = plsc.unpack(acc, format=plsc.PackFormat.INTERLEAVED)
                buf_out[pl.ds(r * CHUNK + off, L)] = oa
                buf_out[pl.ds(r * CHUNK + off + L, L)] = ob

    n_pairs = n_chunks // 2
    issue_in(0, xa, sia)
    issue_in(1, xb, sib)

    def pair_body(k, carry):
        drain_in(xa, sia)

        @pl.when(k > 0)
        def _():
            drain_out(ya, soa)

        compute(xa, ya)
        issue_out(2 * k, ya, soa)

        @pl.when(k + 1 < n_pairs)
        def _():
            issue_in(2 * k + 2, xa, sia)

        drain_in(xb, sib)

        @pl.when(k > 0)
        def _():
            drain_out(yb, sob)

        compute(xb, yb)
        issue_out(2 * k + 1, yb, sob)

        @pl.when(k + 1 < n_pairs)
        def _():
            issue_in(2 * k + 3, xb, sib)

        return carry

    lax.fori_loop(0, n_pairs, pair_body, 0)
    drain_out(ya, soa)
    drain_out(yb, sob)


def kernel(x, W):
    n = x.shape[0]
    xt = x.transpose(1, 2, 3, 0).reshape(16 * n)
    wf = jnp.concatenate([W.reshape(-1), jnp.zeros((7,), jnp.float32)])
    mesh = plsc.VectorSubcoreMesh(core_axis_name="c", subcore_axis_name="s")
    out = pl.kernel(
        _sc_body,
        out_type=jax.ShapeDtypeStruct((16 * n,), jnp.float32),
        mesh=mesh,
        compiler_params=pltpu.CompilerParams(needs_layout_passes=False),
        scratch_types=[
            pltpu.VMEM((16 * CHUNK,), jnp.float32),
            pltpu.VMEM((16 * CHUNK,), jnp.float32),
            pltpu.VMEM((16 * CHUNK,), jnp.float32),
            pltpu.VMEM((16 * CHUNK,), jnp.float32),
            pltpu.VMEM((L,), jnp.float32),
            pltpu.VMEM((9 * L,), jnp.float32),
            pltpu.SemaphoreType.DMA,
            pltpu.SemaphoreType.DMA,
            pltpu.SemaphoreType.DMA,
            pltpu.SemaphoreType.DMA,
        ],
    )(xt, wf)
    return out.reshape(4, 4, 1, n).transpose(3, 0, 1, 2)


# speedup vs baseline: 1.6327x; 1.6327x over previous
import jax
import jax.numpy as jnp
from jax import lax
from jax.experimental import pallas as pl
from jax.experimental.pallas import tpu as pltpu
from jax.experimental.pallas import tpu_sc as plsc

L = 16
NC, NS = 2, 16
NW = NC * NS
CHUNK = 512
DEPTH = 4


def _tap_table():
    taps = []
    for i in range(4):
        for j in range(4):
            lst = []
            for u in range(3):
                for v in range(3):
                    ii, jj = i + u - 1, j + v - 1
                    if 0 <= ii < 4 and 0 <= jj < 4:
                        lst.append((ii * 4 + jj, u * 3 + v))
            taps.append(lst)
    return taps


_TAPS = _tap_table()


def _sc_body(x_hbm, w_hbm, out_hbm,
             x0, x1, x2, x3, y0, y1, y2, y3, wv,
             si0, si1, si2, si3, so0, so1, so2, so3):
    xs_bufs = [x0, x1, x2, x3]
    ys_bufs = [y0, y1, y2, y3]
    sis = [si0, si1, si2, si3]
    sos = [so0, so1, so2, so3]
    c = lax.axis_index("c")
    s = lax.axis_index("s")
    wid = s * NC + c
    n = x_hbm.shape[0] // L
    tiles_per_worker = n // NW
    n_chunks = tiles_per_worker // CHUNK

    pltpu.sync_copy(w_hbm, wv)
    w16 = wv[...]

    def bcast_lane(vec, k):
        return lax.gather(
            vec,
            jnp.full((L, 1), k, jnp.int32),
            lax.GatherDimensionNumbers(
                offset_dims=(), collapsed_slice_dims=(0,), start_index_map=(0,)
            ),
            slice_sizes=(1,),
            mode=lax.GatherScatterMode.PROMISE_IN_BOUNDS,
        )

    wvecs = [bcast_lane(w16, k) for k in range(9)]
    wb = [plsc.pack(w, w, format=plsc.PackFormat.INTERLEAVED) for w in wvecs]
    zb = jnp.zeros((2 * L,), jnp.bfloat16)

    start = wid * tiles_per_worker

    def issue_in(ci, buf, s_in):
        base = start + ci * CHUNK
        for r in range(L):
            pltpu.async_copy(
                x_hbm.at[pl.ds(r * n + base, CHUNK)],
                buf.at[pl.ds(r * CHUNK, CHUNK)],
                s_in,
            )

    def drain_in(buf, s_in):
        pltpu.make_async_copy(x_hbm.at[pl.ds(0, L * CHUNK)], buf, s_in).wait()

    def issue_out(ci, buf, s_out):
        base = start + ci * CHUNK
        for r in range(L):
            pltpu.async_copy(
                buf.at[pl.ds(r * CHUNK, CHUNK)],
                out_hbm.at[pl.ds(r * n + base, CHUNK)],
                s_out,
            )

    def drain_out(buf, s_out):
        pltpu.make_async_copy(buf, out_hbm.at[pl.ds(0, L * CHUNK)], s_out).wait()

    def compute(buf_in, buf_out):
        @plsc.parallel_loop(0, CHUNK // (2 * L), 1, unroll=1)
        def group_body(g):
            off = g * (2 * L)
            xb = []
            for r in range(L):
                a = buf_in[pl.ds(r * CHUNK + off, L)]
                b = buf_in[pl.ds(r * CHUNK + off + L, L)]
                xb.append(plsc.pack(a, b, format=plsc.PackFormat.INTERLEAVED))
            for r in range(L):
                acc = None
                for (rs, widx) in _TAPS[r]:
                    term = wb[widx] * xb[rs]
                    acc = term if acc is None else acc + term
                acc = jnp.where(xb[r] == zb, zb, acc)
                oa, ob = plsc.unpack(acc, format=plsc.PackFormat.INTERLEAVED)
                buf_out[pl.ds(r * CHUNK + off, L)] = oa
                buf_out[pl.ds(r * CHUNK + off + L, L)] = ob

    n_rounds = n_chunks // DEPTH
    for j in range(DEPTH):
        issue_in(j, xs_bufs[j], sis[j])

    def round_body(k, carry):
        for j in range(DEPTH):
            drain_in(xs_bufs[j], sis[j])

            @pl.when(k > 0)
            def _():
                drain_out(ys_bufs[j], sos[j])

            compute(xs_bufs[j], ys_bufs[j])
            issue_out(DEPTH * k + j, ys_bufs[j], sos[j])

            @pl.when(k + 1 < n_rounds)
            def _():
                issue_in(DEPTH * (k + 1) + j, xs_bufs[j], sis[j])

        return carry

    lax.fori_loop(0, n_rounds, round_body, 0)
    for j in range(DEPTH):
        drain_out(ys_bufs[j], sos[j])


def kernel(x, W):
    n = x.shape[0]
    xt = x.transpose(1, 2, 3, 0).reshape(16 * n)
    wf = jnp.concatenate([W.reshape(-1), jnp.zeros((7,), jnp.float32)])
    mesh = plsc.VectorSubcoreMesh(core_axis_name="c", subcore_axis_name="s")
    out = pl.kernel(
        _sc_body,
        out_type=jax.ShapeDtypeStruct((16 * n,), jnp.float32),
        mesh=mesh,
        compiler_params=pltpu.CompilerParams(needs_layout_passes=False),
        scratch_types=(
            [pltpu.VMEM((16 * CHUNK,), jnp.float32) for _ in range(8)]
            + [pltpu.VMEM((L,), jnp.float32)]
            + [pltpu.SemaphoreType.DMA for _ in range(8)]
        ),
    )(xt, wf)
    return out.reshape(4, 4, 1, n).transpose(3, 0, 1, 2)


# speedup vs baseline: 1.7596x; 1.0777x over previous
import jax
import jax.numpy as jnp
from jax import lax
from jax.experimental import pallas as pl
from jax.experimental.pallas import tpu as pltpu
from jax.experimental.pallas import tpu_sc as plsc

L = 16
NC, NS = 2, 16
NW = NC * NS
CHUNK = 1024
DEPTH = 4


def _tap_table():
    taps = []
    for i in range(4):
        for j in range(4):
            lst = []
            for u in range(3):
                for v in range(3):
                    ii, jj = i + u - 1, j + v - 1
                    if 0 <= ii < 4 and 0 <= jj < 4:
                        lst.append((ii * 4 + jj, u * 3 + v))
            taps.append(lst)
    return taps


_TAPS = _tap_table()


def _sc_body(x_hbm, w_hbm, out_hbm,
             x0, x1, x2, x3, y0, y1, wv,
             si0, si1, si2, si3, so0, so1):
    xs_bufs = [x0, x1, x2, x3]
    ys_bufs = [y0, y1]
    sis = [si0, si1, si2, si3]
    sos = [so0, so1]
    c = lax.axis_index("c")
    s = lax.axis_index("s")
    wid = s * NC + c
    n = x_hbm.shape[0] // L
    tiles_per_worker = n // NW
    n_chunks = tiles_per_worker // CHUNK

    pltpu.sync_copy(w_hbm, wv)
    w16 = wv[...]

    def bcast_lane(vec, k):
        return lax.gather(
            vec,
            jnp.full((L, 1), k, jnp.int32),
            lax.GatherDimensionNumbers(
                offset_dims=(), collapsed_slice_dims=(0,), start_index_map=(0,)
            ),
            slice_sizes=(1,),
            mode=lax.GatherScatterMode.PROMISE_IN_BOUNDS,
        )

    wvecs = [bcast_lane(w16, k) for k in range(9)]
    wb = [plsc.pack(w, w, format=plsc.PackFormat.INTERLEAVED) for w in wvecs]
    zb = jnp.zeros((2 * L,), jnp.bfloat16)

    start = wid * tiles_per_worker

    def issue_in(ci, buf, s_in):
        base = start + ci * CHUNK
        for r in range(L):
            pltpu.async_copy(
                x_hbm.at[pl.ds(r * n + base, CHUNK)],
                buf.at[pl.ds(r * CHUNK, CHUNK)],
                s_in,
            )

    def drain_in(buf, s_in):
        pltpu.make_async_copy(x_hbm.at[pl.ds(0, L * CHUNK)], buf, s_in).wait()

    def issue_out(ci, buf, s_out):
        base = start + ci * CHUNK
        for r in range(L):
            pltpu.async_copy(
                buf.at[pl.ds(r * CHUNK, CHUNK)],
                out_hbm.at[pl.ds(r * n + base, CHUNK)],
                s_out,
            )

    def drain_out(buf, s_out):
        pltpu.make_async_copy(buf, out_hbm.at[pl.ds(0, L * CHUNK)], s_out).wait()

    def compute(buf_in, buf_out):
        @plsc.parallel_loop(0, CHUNK // (2 * L), 1, unroll=1)
        def group_body(g):
            off = g * (2 * L)
            xb = []
            for r in range(L):
                a = buf_in[pl.ds(r * CHUNK + off, L)]
                b = buf_in[pl.ds(r * CHUNK + off + L, L)]
                xb.append(plsc.pack(a, b, format=plsc.PackFormat.INTERLEAVED))
            for r in range(L):
                acc = None
                for (rs, widx) in _TAPS[r]:
                    term = wb[widx] * xb[rs]
                    acc = term if acc is None else acc + term
                acc = jnp.where(xb[r] == zb, zb, acc)
                oa, ob = plsc.unpack(acc, format=plsc.PackFormat.INTERLEAVED)
                buf_out[pl.ds(r * CHUNK + off, L)] = oa
                buf_out[pl.ds(r * CHUNK + off + L, L)] = ob

    n_rounds = n_chunks // DEPTH
    for j in range(DEPTH):
        issue_in(j, xs_bufs[j], sis[j])

    def round_body(k, carry):
        for j in range(DEPTH):
            drain_in(xs_bufs[j], sis[j])
            yj, soj = ys_bufs[j % 2], sos[j % 2]
            if j < 2:
                @pl.when(k > 0)
                def _():
                    drain_out(yj, soj)
            else:
                drain_out(yj, soj)

            compute(xs_bufs[j], yj)
            issue_out(DEPTH * k + j, yj, soj)

            @pl.when(k + 1 < n_rounds)
            def _():
                issue_in(DEPTH * (k + 1) + j, xs_bufs[j], sis[j])

        return carry

    lax.fori_loop(0, n_rounds, round_body, 0)
    drain_out(ys_bufs[0], sos[0])
    drain_out(ys_bufs[1], sos[1])


def kernel(x, W):
    n = x.shape[0]
    xt = x.transpose(1, 2, 3, 0).reshape(16 * n)
    wf = jnp.concatenate([W.reshape(-1), jnp.zeros((7,), jnp.float32)])
    mesh = plsc.VectorSubcoreMesh(core_axis_name="c", subcore_axis_name="s")
    out = pl.kernel(
        _sc_body,
        out_type=jax.ShapeDtypeStruct((16 * n,), jnp.float32),
        mesh=mesh,
        compiler_params=pltpu.CompilerParams(needs_layout_passes=False),
        scratch_types=(
            [pltpu.VMEM((16 * CHUNK,), jnp.float32) for _ in range(6)]
            + [pltpu.VMEM((L,), jnp.float32)]
            + [pltpu.SemaphoreType.DMA for _ in range(6)]
        ),
    )(xt, wf)
    return out.reshape(4, 4, 1, n).transpose(3, 0, 1, 2)


# speedup vs baseline: 1.8094x; 1.0283x over previous
import jax
import jax.numpy as jnp
from jax import lax
from jax.experimental import pallas as pl
from jax.experimental.pallas import tpu as pltpu
from jax.experimental.pallas import tpu_sc as plsc

L = 16
NC, NS = 2, 16
NW = NC * NS
CHUNK = 1024


def _tap_table():
    taps = []
    for i in range(4):
        for j in range(4):
            lst = []
            for u in range(3):
                for v in range(3):
                    ii, jj = i + u - 1, j + v - 1
                    if 0 <= ii < 4 and 0 <= jj < 4:
                        lst.append((ii * 4 + jj, u * 3 + v))
            taps.append(lst)
    return taps


_TAPS = _tap_table()


def _sc_body(x_hbm, w_hbm, out_hbm, xa, xb, ya, yb, wv, sia, sib, soa, sob):
    c = lax.axis_index("c")
    s = lax.axis_index("s")
    wid = s * NC + c
    n = x_hbm.shape[0] // L
    tiles_per_worker = n // NW
    n_chunks = tiles_per_worker // CHUNK

    pltpu.sync_copy(w_hbm, wv)
    w16 = wv[...]

    def bcast_lane(vec, k):
        return lax.gather(
            vec,
            jnp.full((L, 1), k, jnp.int32),
            lax.GatherDimensionNumbers(
                offset_dims=(), collapsed_slice_dims=(0,), start_index_map=(0,)
            ),
            slice_sizes=(1,),
            mode=lax.GatherScatterMode.PROMISE_IN_BOUNDS,
        )

    wvecs = [bcast_lane(w16, k) for k in range(9)]
    wb = [plsc.pack(w, w, format=plsc.PackFormat.INTERLEAVED) for w in wvecs]
    zb = jnp.zeros((2 * L,), jnp.bfloat16)

    start = wid * tiles_per_worker

    def issue_in(ci, buf, s_in):
        base = start + ci * CHUNK
        for r in range(L):
            pltpu.async_copy(
                x_hbm.at[pl.ds(r * n + base, CHUNK)],
                buf.at[pl.ds(r * CHUNK, CHUNK)],
                s_in,
            )

    def drain_in(buf, s_in):
        pltpu.make_async_copy(x_hbm.at[pl.ds(0, L * CHUNK)], buf, s_in).wait()

    def issue_out(ci, buf, s_out):
        base = start + ci * CHUNK
        for r in range(L):
            pltpu.async_copy(
                buf.at[pl.ds(r * CHUNK, CHUNK)],
                out_hbm.at[pl.ds(r * n + base, CHUNK)],
                s_out,
            )

    def drain_out(buf, s_out):
        pltpu.make_async_copy(buf, out_hbm.at[pl.ds(0, L * CHUNK)], s_out).wait()

    def compute(buf_in, buf_out):
        @plsc.parallel_loop(0, CHUNK // (2 * L), 1, unroll=1)
        def group_body(g):
            off = g * (2 * L)
            xb = []
            for r in range(L):
                a = buf_in[pl.ds(r * CHUNK + off, L)]
                b = buf_in[pl.ds(r * CHUNK + off + L, L)]
                xb.append(plsc.pack(a, b, format=plsc.PackFormat.INTERLEAVED))
            for r in range(L):
                acc = None
                for (rs, widx) in _TAPS[r]:
                    term = wb[widx] * xb[rs]
                    acc = term if acc is None else acc + term
                acc = jnp.where(xb[r] == zb, zb, acc)
                oa, ob = plsc.unpack(acc, format=plsc.PackFormat.INTERLEAVED)
                buf_out[pl.ds(r * CHUNK + off, L)] = oa
                buf_out[pl.ds(r * CHUNK + off + L, L)] = ob

    n_pairs = n_chunks // 2
    issue_in(0, xa, sia)
    issue_in(1, xb, sib)

    def pair_body(k, carry):
        drain_in(xa, sia)

        @pl.when(k > 0)
        def _():
            drain_out(ya, soa)

        compute(xa, ya)
        issue_out(2 * k, ya, soa)

        @pl.when(k + 1 < n_pairs)
        def _():
            issue_in(2 * k + 2, xa, sia)

        drain_in(xb, sib)

        @pl.when(k > 0)
        def _():
            drain_out(yb, sob)

        compute(xb, yb)
        issue_out(2 * k + 1, yb, sob)

        @pl.when(k + 1 < n_pairs)
        def _():
            issue_in(2 * k + 3, xb, sib)

        return carry

    lax.fori_loop(0, n_pairs, pair_body, 0)
    drain_out(ya, soa)
    drain_out(yb, sob)


def kernel(x, W):
    n = x.shape[0]
    xt = x.transpose(1, 2, 3, 0).reshape(16 * n)
    wf = jnp.concatenate([W.reshape(-1), jnp.zeros((7,), jnp.float32)])
    mesh = plsc.VectorSubcoreMesh(core_axis_name="c", subcore_axis_name="s")
    out = pl.kernel(
        _sc_body,
        out_type=jax.ShapeDtypeStruct((16 * n,), jnp.float32),
        mesh=mesh,
        compiler_params=pltpu.CompilerParams(needs_layout_passes=False),
        scratch_types=[
            pltpu.VMEM((16 * CHUNK,), jnp.float32),
            pltpu.VMEM((16 * CHUNK,), jnp.float32),
            pltpu.VMEM((16 * CHUNK,), jnp.float32),
            pltpu.VMEM((16 * CHUNK,), jnp.float32),
            pltpu.VMEM((L,), jnp.float32),
            pltpu.SemaphoreType.DMA,
            pltpu.SemaphoreType.DMA,
            pltpu.SemaphoreType.DMA,
            pltpu.SemaphoreType.DMA,
        ],
    )(xt, wf)
    return out.reshape(4, 4, 1, n).transpose(3, 0, 1, 2)
